# baseline (speedup 1.0000x reference)
import jax
import jax.numpy as jnp
from jax.experimental import pallas as pl
from jax.experimental.pallas import tpu as pltpu

B = 128
V = 100000
C = 1024
K = (V + C - 1) // C
TAIL = V - (K - 1) * C
RPS = 8

NEG_BIG = -3.0e38


def _lane_shift_right(x, sh):
    r, w = x.shape
    return jnp.concatenate(
        [jnp.zeros((r, sh), x.dtype), x[:, :w - sh]], axis=1)


def _lane_cumsum(x):
    w = x.shape[1]
    sh = 1
    while sh < w:
        x = x + _lane_shift_right(x, sh)
        sh *= 2
    return x


def _stats_kernel(logits_ref, invt_ref, u_ref, kstar_ref, scal_ref,
                  m_scr, s_scr):
    k = pl.program_id(0)
    x = logits_ref[...]
    invt = invt_ref[...]
    lane = jax.lax.broadcasted_iota(jnp.int32, (B, C), 1)
    hitk = jax.lax.broadcasted_iota(jnp.int32, (B, K), 1) == k

    @pl.when(k < K - 1)
    def _full_chunk():
        scaled = x * invt
        mk = jnp.max(scaled, axis=1, keepdims=True)
        sk = jnp.sum(jnp.exp(scaled - mk), axis=1, keepdims=True)
        m_scr[...] = jnp.where(hitk, mk, m_scr[...])
        s_scr[...] = jnp.where(hitk, sk, s_scr[...])

    @pl.when(k == K - 1)
    def _tail_and_merge():
        scaled = jnp.where(lane < TAIL, x * invt, NEG_BIG)
        mk = jnp.max(scaled, axis=1, keepdims=True)
        sk = jnp.sum(jnp.where(lane < TAIL, jnp.exp(scaled - mk), 0.0),
                     axis=1, keepdims=True)
        mloc = jnp.where(hitk, mk, m_scr[...])
        sloc = jnp.where(hitk, sk, s_scr[...])
        m = jnp.max(mloc, axis=1, keepdims=True)
        a = sloc * jnp.exp(mloc - m)
        prefix = _lane_cumsum(a)
        z = prefix[:, K - 1:K]
        t = u_ref[...] * z
        below = jnp.where(prefix < t, 1.0, 0.0)
        kst = jnp.minimum(jnp.sum(below, axis=1, keepdims=True),
                          float(K - 1))
        kidx = jax.lax.broadcasted_iota(jnp.int32, (B, K), 1).astype(
            jnp.float32)
        pexc = jnp.sum(jnp.where(kidx < kst, a, 0.0), axis=1, keepdims=True)
        kstar_ref[...] = kst.astype(jnp.int32)
        scal_ref[...] = jnp.concatenate(
            [m, t, pexc, kst, jnp.zeros((B, 4), jnp.float32)], axis=1)


def _pick_kernel(kstar_pref, *refs):
    x_refs = refs[:RPS]
    scal_ref, invt_ref, out_ref = refs[RPS:]
    rows = jnp.concatenate(
        [x_refs[j][j:j + 1, :] for j in range(RPS)], axis=0)
    scal = scal_ref[...]
    m = scal[:, 0:1]
    t = scal[:, 1:2]
    pexc = scal[:, 2:3]
    kst = scal[:, 3:4]
    invt = invt_ref[...]
    col = kst * float(C) + jax.lax.broadcasted_iota(
        jnp.int32, (RPS, C), 1).astype(jnp.float32)
    valid = col < float(V)
    e = jnp.where(valid, jnp.exp(rows * invt - m), 0.0)
    prefix = pexc + _lane_cumsum(e)
    cnt = jnp.sum(jnp.where(prefix < t, 1.0, 0.0), axis=1, keepdims=True)
    sample = jnp.minimum(kst * float(C) + cnt, float(V - 1))
    out_ref[...] = jnp.broadcast_to(sample.astype(jnp.int32), (RPS, 128))


def kernel(logits, temperatures):
    u = jax.random.uniform(jax.random.key(42), (B, 1), dtype=jnp.float32)
    invt = (1.0 / temperatures).reshape(B, 1)

    kstar, scal = pl.pallas_call(
        _stats_kernel,
        grid=(K,),
        in_specs=[
            pl.BlockSpec((B, C), lambda k: (0, k)),
            pl.BlockSpec((B, 1), lambda k: (0, 0)),
            pl.BlockSpec((B, 1), lambda k: (0, 0)),
        ],
        out_specs=[
            pl.BlockSpec((B, 1), lambda k: (0, 0)),
            pl.BlockSpec((B, 8), lambda k: (0, 0)),
        ],
        out_shape=[
            jax.ShapeDtypeStruct((B, 1), jnp.int32),
            jax.ShapeDtypeStruct((B, 8), jnp.float32),
        ],
        scratch_shapes=[
            pltpu.VMEM((B, K), jnp.float32),
            pltpu.VMEM((B, K), jnp.float32),
        ],
    )(logits, invt, u)

    in_specs = []
    for j in range(RPS):
        in_specs.append(pl.BlockSpec(
            (RPS, C), lambda i, ks, j=j: (i, ks[i * RPS + j])))
    in_specs.append(pl.BlockSpec((RPS, 8), lambda i, ks: (i, 0)))
    in_specs.append(pl.BlockSpec((RPS, 1), lambda i, ks: (i, 0)))

    out = pl.pallas_call(
        _pick_kernel,
        grid_spec=pltpu.PrefetchScalarGridSpec(
            num_scalar_prefetch=1,
            grid=(B // RPS,),
            in_specs=in_specs,
            out_specs=pl.BlockSpec((RPS, 128), lambda i, ks: (i, 0)),
        ),
        out_shape=jax.ShapeDtypeStruct((B, 128), jnp.int32),
    )(kstar.reshape(B), *([logits] * RPS), scal, invt)

    return out[:, 0]


# speedup vs baseline: 1.2379x; 1.2379x over previous
import jax
import jax.numpy as jnp
from jax.experimental import pallas as pl
from jax.experimental.pallas import tpu as pltpu

B = 128
V = 100000
C = 2048
K = (V + C - 1) // C
TAIL = V - (K - 1) * C
RPS = 8

NEG_BIG = -3.0e38


def _lane_shift_right(x, sh):
    r, w = x.shape
    return jnp.concatenate(
        [jnp.zeros((r, sh), x.dtype), x[:, :w - sh]], axis=1)


def _lane_cumsum(x):
    w = x.shape[1]
    sh = 1
    while sh < w:
        x = x + _lane_shift_right(x, sh)
        sh *= 2
    return x


def _stats_kernel(logits_ref, invt_ref, u_ref, kstar_ref, scal_ref,
                  m_scr, s_scr):
    k = pl.program_id(0)
    x = logits_ref[...]
    invt = invt_ref[...]
    lane = jax.lax.broadcasted_iota(jnp.int32, (B, C), 1)
    hitk = jax.lax.broadcasted_iota(jnp.int32, (B, K), 1) == k

    @pl.when(k < K - 1)
    def _full_chunk():
        scaled = x * invt
        mk = jnp.max(scaled, axis=1, keepdims=True)
        sk = jnp.sum(jnp.exp(scaled - mk), axis=1, keepdims=True)
        m_scr[...] = jnp.where(hitk, mk, m_scr[...])
        s_scr[...] = jnp.where(hitk, sk, s_scr[...])

    @pl.when(k == K - 1)
    def _tail_and_merge():
        scaled = jnp.where(lane < TAIL, x * invt, NEG_BIG)
        mk = jnp.max(scaled, axis=1, keepdims=True)
        sk = jnp.sum(jnp.where(lane < TAIL, jnp.exp(scaled - mk), 0.0),
                     axis=1, keepdims=True)
        mloc = jnp.where(hitk, mk, m_scr[...])
        sloc = jnp.where(hitk, sk, s_scr[...])
        m = jnp.max(mloc, axis=1, keepdims=True)
        a = sloc * jnp.exp(mloc - m)
        prefix = _lane_cumsum(a)
        z = prefix[:, K - 1:K]
        t = u_ref[...] * z
        below = jnp.where(prefix < t, 1.0, 0.0)
        kst = jnp.minimum(jnp.sum(below, axis=1, keepdims=True),
                          float(K - 1))
        kidx = jax.lax.broadcasted_iota(jnp.int32, (B, K), 1).astype(
            jnp.float32)
        pexc = jnp.sum(jnp.where(kidx < kst, a, 0.0), axis=1, keepdims=True)
        kstar_ref[...] = kst.astype(jnp.int32)
        scal_ref[...] = jnp.concatenate(
            [m, t, pexc, kst, jnp.zeros((B, 4), jnp.float32)], axis=1)


def _pick_kernel(kstar_pref, *refs):
    x_refs = refs[:RPS]
    scal_ref, invt_ref, out_ref = refs[RPS:]
    rows = jnp.concatenate(
        [x_refs[j][j:j + 1, :] for j in range(RPS)], axis=0)
    scal = scal_ref[...]
    m = scal[:, 0:1]
    t = scal[:, 1:2]
    pexc = scal[:, 2:3]
    kst = scal[:, 3:4]
    invt = invt_ref[...]
    col = kst * float(C) + jax.lax.broadcasted_iota(
        jnp.int32, (RPS, C), 1).astype(jnp.float32)
    valid = col < float(V)
    e = jnp.where(valid, jnp.exp(rows * invt - m), 0.0)
    prefix = pexc + _lane_cumsum(e)
    cnt = jnp.sum(jnp.where(prefix < t, 1.0, 0.0), axis=1, keepdims=True)
    sample = jnp.minimum(kst * float(C) + cnt, float(V - 1))
    out_ref[...] = jnp.broadcast_to(sample.astype(jnp.int32), (RPS, 128))


def kernel(logits, temperatures):
    u = jax.random.uniform(jax.random.key(42), (B, 1), dtype=jnp.float32)
    invt = (1.0 / temperatures).reshape(B, 1)

    kstar, scal = pl.pallas_call(
        _stats_kernel,
        grid=(K,),
        in_specs=[
            pl.BlockSpec((B, C), lambda k: (0, k)),
            pl.BlockSpec((B, 1), lambda k: (0, 0)),
            pl.BlockSpec((B, 1), lambda k: (0, 0)),
        ],
        out_specs=[
            pl.BlockSpec((B, 1), lambda k: (0, 0)),
            pl.BlockSpec((B, 8), lambda k: (0, 0)),
        ],
        out_shape=[
            jax.ShapeDtypeStruct((B, 1), jnp.int32),
            jax.ShapeDtypeStruct((B, 8), jnp.float32),
        ],
        scratch_shapes=[
            pltpu.VMEM((B, K), jnp.float32),
            pltpu.VMEM((B, K), jnp.float32),
        ],
    )(logits, invt, u)

    in_specs = []
    for j in range(RPS):
        in_specs.append(pl.BlockSpec(
            (RPS, C), lambda i, ks, j=j: (i, ks[i * RPS + j])))
    in_specs.append(pl.BlockSpec((RPS, 8), lambda i, ks: (i, 0)))
    in_specs.append(pl.BlockSpec((RPS, 1), lambda i, ks: (i, 0)))

    out = pl.pallas_call(
        _pick_kernel,
        grid_spec=pltpu.PrefetchScalarGridSpec(
            num_scalar_prefetch=1,
            grid=(B // RPS,),
            in_specs=in_specs,
            out_specs=pl.BlockSpec((RPS, 128), lambda i, ks: (i, 0)),
        ),
        out_shape=jax.ShapeDtypeStruct((B, 128), jnp.int32),
    )(kstar.reshape(B), *([logits] * RPS), scal, invt)

    return out[:, 0]


# speedup vs baseline: 1.3822x; 1.1166x over previous
import jax
import jax.numpy as jnp
from jax.experimental import pallas as pl
from jax.experimental.pallas import tpu as pltpu

B = 128
V = 100000
C = 4096
K = (V + C - 1) // C
TAIL = V - (K - 1) * C
RPS = 8

NEG_BIG = -3.0e38


def _lane_shift_right(x, sh):
    r, w = x.shape
    return jnp.concatenate(
        [jnp.zeros((r, sh), x.dtype), x[:, :w - sh]], axis=1)


def _lane_cumsum(x):
    w = x.shape[1]
    sh = 1
    while sh < w:
        x = x + _lane_shift_right(x, sh)
        sh *= 2
    return x


def _stats_kernel(logits_ref, invt_ref, u_ref, kstar_ref, scal_ref,
                  m_scr, s_scr):
    k = pl.program_id(0)
    x = logits_ref[...]
    invt = invt_ref[...]
    lane = jax.lax.broadcasted_iota(jnp.int32, (B, C), 1)
    hitk = jax.lax.broadcasted_iota(jnp.int32, (B, K), 1) == k

    @pl.when(k < K - 1)
    def _full_chunk():
        scaled = x * invt
        mk = jnp.max(scaled, axis=1, keepdims=True)
        sk = jnp.sum(jnp.exp(scaled - mk), axis=1, keepdims=True)
        m_scr[...] = jnp.where(hitk, mk, m_scr[...])
        s_scr[...] = jnp.where(hitk, sk, s_scr[...])

    @pl.when(k == K - 1)
    def _tail_and_merge():
        scaled = jnp.where(lane < TAIL, x * invt, NEG_BIG)
        mk = jnp.max(scaled, axis=1, keepdims=True)
        sk = jnp.sum(jnp.where(lane < TAIL, jnp.exp(scaled - mk), 0.0),
                     axis=1, keepdims=True)
        mloc = jnp.where(hitk, mk, m_scr[...])
        sloc = jnp.where(hitk, sk, s_scr[...])
        m = jnp.max(mloc, axis=1, keepdims=True)
        a = sloc * jnp.exp(mloc - m)
        prefix = _lane_cumsum(a)
        z = prefix[:, K - 1:K]
        t = u_ref[...] * z
        below = jnp.where(prefix < t, 1.0, 0.0)
        kst = jnp.minimum(jnp.sum(below, axis=1, keepdims=True),
                          float(K - 1))
        kidx = jax.lax.broadcasted_iota(jnp.int32, (B, K), 1).astype(
            jnp.float32)
        pexc = jnp.sum(jnp.where(kidx < kst, a, 0.0), axis=1, keepdims=True)
        kstar_ref[...] = kst.astype(jnp.int32)
        scal_ref[...] = jnp.concatenate(
            [m, t, pexc, kst, jnp.zeros((B, 4), jnp.float32)], axis=1)


def _pick_kernel(kstar_pref, *refs):
    x_refs = refs[:RPS]
    scal_ref, invt_ref, out_ref = refs[RPS:]
    rows = jnp.concatenate(
        [x_refs[j][j:j + 1, :] for j in range(RPS)], axis=0)
    scal = scal_ref[...]
    m = scal[:, 0:1]
    t = scal[:, 1:2]
    pexc = scal[:, 2:3]
    kst = scal[:, 3:4]
    invt = invt_ref[...]
    col = kst * float(C) + jax.lax.broadcasted_iota(
        jnp.int32, (RPS, C), 1).astype(jnp.float32)
    valid = col < float(V)
    e = jnp.where(valid, jnp.exp(rows * invt - m), 0.0)
    prefix = pexc + _lane_cumsum(e)
    cnt = jnp.sum(jnp.where(prefix < t, 1.0, 0.0), axis=1, keepdims=True)
    sample = jnp.minimum(kst * float(C) + cnt, float(V - 1))
    out_ref[...] = jnp.broadcast_to(sample.astype(jnp.int32), (RPS, 128))


def kernel(logits, temperatures):
    u = jax.random.uniform(jax.random.key(42), (B, 1), dtype=jnp.float32)
    invt = (1.0 / temperatures).reshape(B, 1)

    kstar, scal = pl.pallas_call(
        _stats_kernel,
        grid=(K,),
        in_specs=[
            pl.BlockSpec((B, C), lambda k: (0, k)),
            pl.BlockSpec((B, 1), lambda k: (0, 0)),
            pl.BlockSpec((B, 1), lambda k: (0, 0)),
        ],
        out_specs=[
            pl.BlockSpec((B, 1), lambda k: (0, 0)),
            pl.BlockSpec((B, 8), lambda k: (0, 0)),
        ],
        out_shape=[
            jax.ShapeDtypeStruct((B, 1), jnp.int32),
            jax.ShapeDtypeStruct((B, 8), jnp.float32),
        ],
        scratch_shapes=[
            pltpu.VMEM((B, K), jnp.float32),
            pltpu.VMEM((B, K), jnp.float32),
        ],
    )(logits, invt, u)

    in_specs = []
    for j in range(RPS):
        in_specs.append(pl.BlockSpec(
            (RPS, C), lambda i, ks, j=j: (i, ks[i * RPS + j])))
    in_specs.append(pl.BlockSpec((RPS, 8), lambda i, ks: (i, 0)))
    in_specs.append(pl.BlockSpec((RPS, 1), lambda i, ks: (i, 0)))

    out = pl.pallas_call(
        _pick_kernel,
        grid_spec=pltpu.PrefetchScalarGridSpec(
            num_scalar_prefetch=1,
            grid=(B // RPS,),
            in_specs=in_specs,
            out_specs=pl.BlockSpec((RPS, 128), lambda i, ks: (i, 0)),
        ),
        out_shape=jax.ShapeDtypeStruct((B, 128), jnp.int32),
    )(kstar.reshape(B), *([logits] * RPS), scal, invt)

    return out[:, 0]


# speedup vs baseline: 1.5519x; 1.1227x over previous
import jax
import jax.numpy as jnp
from jax.experimental import pallas as pl
from jax.experimental.pallas import tpu as pltpu

B = 128
V = 100000
C = 4096
K = (V + C - 1) // C
TAIL = V - (K - 1) * C
RPS = 8

NEG_BIG = -3.0e38


def _lane_shift_right(x, sh):
    r, w = x.shape
    return jnp.concatenate(
        [jnp.zeros((r, sh), x.dtype), x[:, :w - sh]], axis=1)


def _lane_cumsum(x):
    w = x.shape[1]
    sh = 1
    while sh < w:
        x = x + _lane_shift_right(x, sh)
        sh *= 2
    return x


def _stats_kernel(logits_ref, invt_ref, u_ref, kstar_ref, scal_ref,
                  m_scr, s_scr):
    k = pl.program_id(0)
    x = logits_ref[...]
    invt = invt_ref[...]
    lane = jax.lax.broadcasted_iota(jnp.int32, (B, C), 1)
    hitk = jax.lax.broadcasted_iota(jnp.int32, (B, K), 1) == k

    @pl.when(k < K - 1)
    def _full_chunk():
        scaled = x * invt
        mk = jnp.max(scaled, axis=1, keepdims=True)
        sk = jnp.sum(jnp.exp(scaled - mk), axis=1, keepdims=True)
        m_scr[...] = jnp.where(hitk, mk, m_scr[...])
        s_scr[...] = jnp.where(hitk, sk, s_scr[...])

    @pl.when(k == K - 1)
    def _tail_and_merge():
        scaled = jnp.where(lane < TAIL, x * invt, NEG_BIG)
        mk = jnp.max(scaled, axis=1, keepdims=True)
        sk = jnp.sum(jnp.where(lane < TAIL, jnp.exp(scaled - mk), 0.0),
                     axis=1, keepdims=True)
        mloc = jnp.where(hitk, mk, m_scr[...])
        sloc = jnp.where(hitk, sk, s_scr[...])
        m = jnp.max(mloc, axis=1, keepdims=True)
        a = sloc * jnp.exp(mloc - m)
        prefix = _lane_cumsum(a)
        z = prefix[:, K - 1:K]
        t = u_ref[...] * z
        below = jnp.where(prefix < t, 1.0, 0.0)
        kst = jnp.minimum(jnp.sum(below, axis=1, keepdims=True),
                          float(K - 1))
        kidx = jax.lax.broadcasted_iota(jnp.int32, (B, K), 1).astype(
            jnp.float32)
        pexc = jnp.sum(jnp.where(kidx < kst, a, 0.0), axis=1, keepdims=True)
        kstar_ref[...] = kst.astype(jnp.int32)
        scal_ref[...] = jnp.concatenate(
            [m, t, pexc, kst, jnp.zeros((B, 4), jnp.float32)], axis=1)


def _pick_kernel(kstar_ref, logits_ref, scal_ref, invt_ref, out_ref,
                 gbuf, sem):
    copies = []
    for b in range(B):
        start = kstar_ref[b] * C
        cp = pltpu.make_async_copy(
            logits_ref.at[pl.ds(b, 1), pl.ds(start, C)],
            gbuf.at[pl.ds(b, 1), :], sem)
        cp.start()
        copies.append(cp)
    for cp in copies:
        cp.wait()
    scal = scal_ref[...]
    m = scal[:, 0:1]
    t = scal[:, 1:2]
    pexc = scal[:, 2:3]
    kst = scal[:, 3:4]
    invt = invt_ref[...]
    rows = gbuf[...]
    col = kst * float(C) + jax.lax.broadcasted_iota(
        jnp.int32, (B, C), 1).astype(jnp.float32)
    valid = col < float(V)
    e = jnp.where(valid, jnp.exp(rows * invt - m), 0.0)
    prefix = pexc + _lane_cumsum(e)
    cnt = jnp.sum(jnp.where(prefix < t, 1.0, 0.0), axis=1, keepdims=True)
    sample = jnp.minimum(kst * float(C) + cnt, float(V - 1))
    out_ref[...] = jnp.broadcast_to(sample.astype(jnp.int32), (B, 128))


def kernel(logits, temperatures):
    u = jax.random.uniform(jax.random.key(42), (B, 1), dtype=jnp.float32)
    invt = (1.0 / temperatures).reshape(B, 1)

    kstar, scal = pl.pallas_call(
        _stats_kernel,
        grid=(K,),
        in_specs=[
            pl.BlockSpec((B, C), lambda k: (0, k)),
            pl.BlockSpec((B, 1), lambda k: (0, 0)),
            pl.BlockSpec((B, 1), lambda k: (0, 0)),
        ],
        out_specs=[
            pl.BlockSpec((B, 1), lambda k: (0, 0)),
            pl.BlockSpec((B, 8), lambda k: (0, 0)),
        ],
        out_shape=[
            jax.ShapeDtypeStruct((B, 1), jnp.int32),
            jax.ShapeDtypeStruct((B, 8), jnp.float32),
        ],
        scratch_shapes=[
            pltpu.VMEM((B, K), jnp.float32),
            pltpu.VMEM((B, K), jnp.float32),
        ],
    )(logits, invt, u)

    out = pl.pallas_call(
        _pick_kernel,
        in_specs=[
            pl.BlockSpec(memory_space=pltpu.SMEM),
            pl.BlockSpec(memory_space=pl.ANY),
            pl.BlockSpec((B, 8), lambda: (0, 0)),
            pl.BlockSpec((B, 1), lambda: (0, 0)),
        ],
        out_specs=pl.BlockSpec((B, 128), lambda: (0, 0)),
        out_shape=jax.ShapeDtypeStruct((B, 128), jnp.int32),
        scratch_shapes=[
            pltpu.VMEM((B, C), jnp.float32),
            pltpu.SemaphoreType.DMA,
        ],
    )(kstar.reshape(B), logits, scal, invt)

    return out[:, 0]


# speedup vs baseline: 1.5798x; 1.0180x over previous
import jax
import jax.numpy as jnp
from jax.experimental import pallas as pl
from jax.experimental.pallas import tpu as pltpu

B = 128
V = 100000
C = 8192
K = (V + C - 1) // C
TAIL = V - (K - 1) * C
RPS = 8

NEG_BIG = -3.0e38


def _lane_shift_right(x, sh):
    r, w = x.shape
    return jnp.concatenate(
        [jnp.zeros((r, sh), x.dtype), x[:, :w - sh]], axis=1)


def _lane_cumsum(x):
    w = x.shape[1]
    sh = 1
    while sh < w:
        x = x + _lane_shift_right(x, sh)
        sh *= 2
    return x


def _stats_kernel(logits_ref, invt_ref, u_ref, kstar_ref, scal_ref,
                  m_scr, s_scr):
    k = pl.program_id(0)
    x = logits_ref[...]
    invt = invt_ref[...]
    lane = jax.lax.broadcasted_iota(jnp.int32, (B, C), 1)
    hitk = jax.lax.broadcasted_iota(jnp.int32, (B, K), 1) == k

    @pl.when(k < K - 1)
    def _full_chunk():
        scaled = x * invt
        mk = jnp.max(scaled, axis=1, keepdims=True)
        sk = jnp.sum(jnp.exp(scaled - mk), axis=1, keepdims=True)
        m_scr[...] = jnp.where(hitk, mk, m_scr[...])
        s_scr[...] = jnp.where(hitk, sk, s_scr[...])

    @pl.when(k == K - 1)
    def _tail_and_merge():
        scaled = jnp.where(lane < TAIL, x * invt, NEG_BIG)
        mk = jnp.max(scaled, axis=1, keepdims=True)
        sk = jnp.sum(jnp.where(lane < TAIL, jnp.exp(scaled - mk), 0.0),
                     axis=1, keepdims=True)
        mloc = jnp.where(hitk, mk, m_scr[...])
        sloc = jnp.where(hitk, sk, s_scr[...])
        m = jnp.max(mloc, axis=1, keepdims=True)
        a = sloc * jnp.exp(mloc - m)
        prefix = _lane_cumsum(a)
        z = prefix[:, K - 1:K]
        t = u_ref[...] * z
        below = jnp.where(prefix < t, 1.0, 0.0)
        kst = jnp.minimum(jnp.sum(below, axis=1, keepdims=True),
                          float(K - 1))
        kidx = jax.lax.broadcasted_iota(jnp.int32, (B, K), 1).astype(
            jnp.float32)
        pexc = jnp.sum(jnp.where(kidx < kst, a, 0.0), axis=1, keepdims=True)
        kstar_ref[...] = kst.astype(jnp.int32)
        scal_ref[...] = jnp.concatenate(
            [m, t, pexc, kst, jnp.zeros((B, 4), jnp.float32)], axis=1)


def _pick_kernel(kstar_ref, logits_ref, scal_ref, invt_ref, out_ref,
                 gbuf, sem):
    copies = []
    for b in range(B):
        start = kstar_ref[b] * C
        cp = pltpu.make_async_copy(
            logits_ref.at[pl.ds(b, 1), pl.ds(start, C)],
            gbuf.at[pl.ds(b, 1), :], sem)
        cp.start()
        copies.append(cp)
    for cp in copies:
        cp.wait()
    scal = scal_ref[...]
    m = scal[:, 0:1]
    t = scal[:, 1:2]
    pexc = scal[:, 2:3]
    kst = scal[:, 3:4]
    invt = invt_ref[...]
    rows = gbuf[...]
    col = kst * float(C) + jax.lax.broadcasted_iota(
        jnp.int32, (B, C), 1).astype(jnp.float32)
    valid = col < float(V)
    e = jnp.where(valid, jnp.exp(rows * invt - m), 0.0)
    prefix = pexc + _lane_cumsum(e)
    cnt = jnp.sum(jnp.where(prefix < t, 1.0, 0.0), axis=1, keepdims=True)
    sample = jnp.minimum(kst * float(C) + cnt, float(V - 1))
    out_ref[...] = jnp.broadcast_to(sample.astype(jnp.int32), (B, 128))


def kernel(logits, temperatures):
    u = jax.random.uniform(jax.random.key(42), (B, 1), dtype=jnp.float32)
    invt = (1.0 / temperatures).reshape(B, 1)

    kstar, scal = pl.pallas_call(
        _stats_kernel,
        grid=(K,),
        in_specs=[
            pl.BlockSpec((B, C), lambda k: (0, k)),
            pl.BlockSpec((B, 1), lambda k: (0, 0)),
            pl.BlockSpec((B, 1), lambda k: (0, 0)),
        ],
        out_specs=[
            pl.BlockSpec((B, 1), lambda k: (0, 0)),
            pl.BlockSpec((B, 8), lambda k: (0, 0)),
        ],
        out_shape=[
            jax.ShapeDtypeStruct((B, 1), jnp.int32),
            jax.ShapeDtypeStruct((B, 8), jnp.float32),
        ],
        scratch_shapes=[
            pltpu.VMEM((B, K), jnp.float32),
            pltpu.VMEM((B, K), jnp.float32),
        ],
    )(logits, invt, u)

    out = pl.pallas_call(
        _pick_kernel,
        in_specs=[
            pl.BlockSpec(memory_space=pltpu.SMEM),
            pl.BlockSpec(memory_space=pl.ANY),
            pl.BlockSpec((B, 8), lambda: (0, 0)),
            pl.BlockSpec((B, 1), lambda: (0, 0)),
        ],
        out_specs=pl.BlockSpec((B, 128), lambda: (0, 0)),
        out_shape=jax.ShapeDtypeStruct((B, 128), jnp.int32),
        scratch_shapes=[
            pltpu.VMEM((B, C), jnp.float32),
            pltpu.SemaphoreType.DMA,
        ],
    )(kstar.reshape(B), logits, scal, invt)

    return out[:, 0]


# speedup vs baseline: 1.6722x; 1.0585x over previous
import jax
import jax.numpy as jnp
from jax.experimental import pallas as pl
from jax.experimental.pallas import tpu as pltpu

B = 128
V = 100000
F = 1024
SUB = 8
C = F * SUB
K = (V + C - 1) // C
TAIL = V - (K - 1) * C
NFV = (V + F - 1) // F

NEG_BIG = -3.0e38


def _lane_shift_right(x, sh):
    r, w = x.shape
    return jnp.concatenate(
        [jnp.zeros((r, sh), x.dtype), x[:, :w - sh]], axis=1)


def _lane_cumsum(x):
    w = x.shape[1]
    sh = 1
    while sh < w:
        x = x + _lane_shift_right(x, sh)
        sh *= 2
    return x


def _substats(scaled, masked):
    mks, sks = [], []
    for i in range(SUB):
        xs = scaled[:, i * F:(i + 1) * F]
        mk = jnp.max(xs, axis=1, keepdims=True)
        e = jnp.exp(xs - mk)
        if masked:
            lane = jax.lax.broadcasted_iota(jnp.int32, (B, F), 1)
            e = jnp.where(i * F + lane < TAIL, e, 0.0)
        sks.append(jnp.sum(e, axis=1, keepdims=True))
        mks.append(mk)
    return jnp.concatenate(mks, axis=1), jnp.concatenate(sks, axis=1)


def _stats_kernel(logits_ref, invt_ref, u_ref, start_ref, scal_ref,
                  m3, s3):
    k = pl.program_id(0)
    x = logits_ref[...]
    invt = invt_ref[...]

    @pl.when(k < K - 1)
    def _full_block():
        mk, sk = _substats(x * invt, masked=False)
        m3[pl.ds(k, 1)] = mk.reshape(1, B, SUB)
        s3[pl.ds(k, 1)] = sk.reshape(1, B, SUB)

    @pl.when(k == K - 1)
    def _tail_and_merge():
        lane = jax.lax.broadcasted_iota(jnp.int32, (B, C), 1)
        scaled = jnp.where(lane < TAIL, x * invt, NEG_BIG)
        mk, sk = _substats(scaled, masked=True)
        m3[pl.ds(k, 1)] = mk.reshape(1, B, SUB)
        s3[pl.ds(k, 1)] = sk.reshape(1, B, SUB)

        m = jnp.full((B, 1), NEG_BIG, jnp.float32)
        for kk in range(K):
            m = jnp.maximum(m, jnp.max(m3[kk], axis=1, keepdims=True))
        z = jnp.zeros((B, 1), jnp.float32)
        for kk in range(K):
            a = s3[kk] * jnp.exp(m3[kk] - m)
            z = z + jnp.sum(a, axis=1, keepdims=True)
        t = u_ref[...] * z
        run = jnp.zeros((B, 1), jnp.float32)
        cnt = jnp.zeros((B, 1), jnp.float32)
        pexc = jnp.zeros((B, 1), jnp.float32)
        for kk in range(K):
            a = s3[kk] * jnp.exp(m3[kk] - m)
            p = run + _lane_cumsum(a)
            below = p < t
            cnt = cnt + jnp.sum(
                jnp.where(below, 1.0, 0.0), axis=1, keepdims=True)
            pexc = pexc + jnp.sum(
                jnp.where(below, a, 0.0), axis=1, keepdims=True)
            run = run + jnp.sum(a, axis=1, keepdims=True)
        kst = jnp.minimum(cnt, float(NFV - 1))
        start_ref[...] = jnp.minimum(kst, float(NFV - 2)).astype(jnp.int32)
        scal_ref[...] = jnp.concatenate(
            [m, t, pexc, kst, jnp.zeros((B, 4), jnp.float32)], axis=1)


def _pick_kernel(kidx_ref, logits_ref, tail_ref, scal_ref, invt_ref,
                 out_ref, gbuf, sem):
    copies = []
    for b in range(B):
        start = kidx_ref[b] * F
        cp = pltpu.make_async_copy(
            logits_ref.at[pl.ds(b, 1), pl.ds(start, F)],
            gbuf.at[pl.ds(b, 1), :], sem)
        cp.start()
        copies.append(cp)
    for cp in copies:
        cp.wait()
    scal = scal_ref[...]
    m = scal[:, 0:1]
    t = scal[:, 1:2]
    pexc = scal[:, 2:3]
    kst = scal[:, 3:4]
    invt = invt_ref[...]
    lastrow = kst >= float(NFV - 1)
    rows = jnp.where(lastrow, tail_ref[...], gbuf[...])
    lane = jax.lax.broadcasted_iota(jnp.int32, (B, F), 1)
    valid = jnp.logical_or(jnp.logical_not(lastrow),
                           lane < V - (NFV - 1) * F)
    e = jnp.where(valid, jnp.exp(rows * invt - m), 0.0)
    prefix = pexc + _lane_cumsum(e)
    cnt = jnp.sum(
        jnp.where(jnp.logical_and(prefix < t, valid), 1.0, 0.0),
        axis=1, keepdims=True)
    sample = jnp.minimum(kst * float(F) + cnt, float(V - 1))
    out_ref[...] = jnp.broadcast_to(sample.astype(jnp.int32), (B, 128))


def kernel(logits, temperatures):
    u = jax.random.uniform(jax.random.key(42), (B, 1), dtype=jnp.float32)
    invt = (1.0 / temperatures).reshape(B, 1)

    start, scal = pl.pallas_call(
        _stats_kernel,
        grid=(K,),
        in_specs=[
            pl.BlockSpec((B, C), lambda k: (0, k)),
            pl.BlockSpec((B, 1), lambda k: (0, 0)),
            pl.BlockSpec((B, 1), lambda k: (0, 0)),
        ],
        out_specs=[
            pl.BlockSpec((B, 1), lambda k: (0, 0)),
            pl.BlockSpec((B, 8), lambda k: (0, 0)),
        ],
        out_shape=[
            jax.ShapeDtypeStruct((B, 1), jnp.int32),
            jax.ShapeDtypeStruct((B, 8), jnp.float32),
        ],
        scratch_shapes=[
            pltpu.VMEM((K, B, SUB), jnp.float32),
            pltpu.VMEM((K, B, SUB), jnp.float32),
        ],
    )(logits, invt, u)

    out = pl.pallas_call(
        _pick_kernel,
        grid=(1,),
        in_specs=[
            pl.BlockSpec(memory_space=pltpu.SMEM),
            pl.BlockSpec(memory_space=pl.ANY),
            pl.BlockSpec((B, F), lambda i: (0, NFV - 1)),
            pl.BlockSpec((B, 8), lambda i: (0, 0)),
            pl.BlockSpec((B, 1), lambda i: (0, 0)),
        ],
        out_specs=pl.BlockSpec((B, 128), lambda i: (0, 0)),
        out_shape=jax.ShapeDtypeStruct((B, 128), jnp.int32),
        scratch_shapes=[
            pltpu.VMEM((B, F), jnp.float32),
            pltpu.SemaphoreType.DMA,
        ],
    )(start.reshape(B), logits, logits, scal, invt)

    return out[:, 0]
